# Initial kernel scaffold; baseline (speedup 1.0000x reference)
import jax
import jax.numpy as jnp
from jax.experimental import pallas as pl


def kernel(x, edge_index, W_l, b_l, W_r, gamma, beta):
    raise NotImplementedError("write your pallas kernel here")


# speedup vs baseline: 4.0488x; 4.0488x over previous
import functools

import jax
import jax.numpy as jnp
from jax import lax
from jax.experimental import pallas as pl
from jax.experimental.pallas import tpu as pltpu
from jax.experimental.pallas import tpu_sc as plsc

N_NODES = 10000
N_EDGES = 320000
D = 128

NC = 2
NS = 16
NW = NC * NS

N_PAD = 10240
TRASH = 10200
ROWS_PER_TILE = N_PAD // NS
CHUNK = 64
NCHUNK = 160
E_PAD = NW * NCHUNK * CHUNK
KB = 16
NKB = NCHUNK // KB


def _sc_aggregate_body(x_hbm, src_hbm, dst_hbm, zrows_hbm, zcnt_hbm, ones_hbm,
                       agg_out, cnt_out,
                       src_v, dst_v, rows_v, ones_v, cstage_v,
                       agg_sh, cnt_sh, sem):
    c = lax.axis_index("c")
    s = lax.axis_index("s")
    wid = s * NC + c

    pltpu.sync_copy(ones_hbm, ones_v)

    base = s * ROWS_PER_TILE
    pltpu.sync_copy(zrows_hbm, rows_v)
    pltpu.sync_copy(zcnt_hbm, cstage_v)
    for j in range(ROWS_PER_TILE // CHUNK):
        pltpu.sync_copy(rows_v, agg_sh.at[pl.ds(base + j * CHUNK, CHUNK)])
    pltpu.sync_copy(cstage_v, cnt_sh.at[pl.ds(base, ROWS_PER_TILE)])
    plsc.subcore_barrier()

    def block_body(kb, carry):
        k0 = pl.multiple_of(kb * KB, KB)
        pltpu.sync_copy(src_hbm.at[wid, pl.ds(k0, KB)], src_v)
        pltpu.sync_copy(dst_hbm.at[wid, pl.ds(k0, KB)], dst_v)

        def chunk_body(ci, carry2):
            pltpu.async_copy(x_hbm.at[src_v.at[ci]], rows_v, sem).wait()
            pltpu.sync_copy(rows_v, agg_sh.at[dst_v.at[ci]], add=True)
            pltpu.sync_copy(ones_v, cnt_sh.at[dst_v.at[ci]], add=True)
            return carry2

        return lax.fori_loop(0, KB, chunk_body, carry)

    lax.fori_loop(0, NKB, block_body, 0)
    plsc.subcore_barrier()

    out_base = c * N_PAD + base
    for j in range(ROWS_PER_TILE // CHUNK):
        pltpu.sync_copy(agg_sh.at[pl.ds(base + j * CHUNK, CHUNK)], rows_v)
        pltpu.sync_copy(rows_v, agg_out.at[pl.ds(out_base + j * CHUNK, CHUNK)])
    pltpu.sync_copy(cnt_sh.at[pl.ds(base, ROWS_PER_TILE)], cstage_v)
    pltpu.sync_copy(cstage_v, cnt_out.at[pl.ds(out_base, ROWS_PER_TILE)])


_sc_aggregate = functools.partial(
    pl.kernel,
    mesh=plsc.VectorSubcoreMesh(core_axis_name="c", subcore_axis_name="s"),
    out_type=[
        jax.ShapeDtypeStruct((NC * N_PAD, D), jnp.float32),
        jax.ShapeDtypeStruct((NC * N_PAD,), jnp.float32),
    ],
    scratch_types=[
        pltpu.VMEM((KB, CHUNK), jnp.int32),
        pltpu.VMEM((KB, CHUNK), jnp.int32),
        pltpu.VMEM((CHUNK, D), jnp.float32),
        pltpu.VMEM((CHUNK,), jnp.float32),
        pltpu.VMEM((ROWS_PER_TILE,), jnp.float32),
        pltpu.VMEM_SHARED((N_PAD, D), jnp.float32),
        pltpu.VMEM_SHARED((N_PAD,), jnp.float32),
        pltpu.SemaphoreType.DMA,
    ],
)(_sc_aggregate_body)


def _tc_block_body(agg_ref, cnt_ref, x_ref, wl_ref, wr_ref, b_ref, g_ref,
                   be_ref, o_ref):
    a = agg_ref[0] + agg_ref[1]
    cn = cnt_ref[0] + cnt_ref[1]
    h = a * (1.0 / jnp.maximum(cn, 1.0))
    xb = x_ref[...]
    f = (jnp.dot(h, wl_ref[...], preferred_element_type=jnp.float32)
         + jnp.dot(xb, wr_ref[...], preferred_element_type=jnp.float32)
         + b_ref[...])
    f = 0.5 * f * (1.0 + lax.erf(f * 0.7071067811865476))
    mean = jnp.mean(f, axis=1, keepdims=True)
    d = f - mean
    var = jnp.mean(d * d, axis=1, keepdims=True)
    f = d * lax.rsqrt(var + 1e-5) * g_ref[...] + be_ref[...]
    o_ref[...] = f + xb


def _tc_tail(agg, cnt, x, wlT, wrT, b, g, be):
    R = 1000
    grid = (N_NODES // R,)
    return pl.pallas_call(
        _tc_block_body,
        grid=grid,
        in_specs=[
            pl.BlockSpec((NC, R, D), lambda i: (0, i, 0)),
            pl.BlockSpec((NC, R, 1), lambda i: (0, i, 0)),
            pl.BlockSpec((R, D), lambda i: (i, 0)),
            pl.BlockSpec((D, D), lambda i: (0, 0)),
            pl.BlockSpec((D, D), lambda i: (0, 0)),
            pl.BlockSpec((1, D), lambda i: (0, 0)),
            pl.BlockSpec((1, D), lambda i: (0, 0)),
            pl.BlockSpec((1, D), lambda i: (0, 0)),
        ],
        out_specs=pl.BlockSpec((R, D), lambda i: (i, 0)),
        out_shape=jax.ShapeDtypeStruct((N_NODES, D), jnp.float32),
    )(agg, cnt, x, wlT, wrT, b, g, be)


def kernel(x, edge_index, W_l, b_l, W_r, gamma, beta):
    npad = E_PAD - N_EDGES
    src = jnp.concatenate(
        [edge_index[0].astype(jnp.int32), jnp.zeros((npad,), jnp.int32)]
    ).reshape(NW, NCHUNK, CHUNK)
    dst = jnp.concatenate(
        [edge_index[1].astype(jnp.int32), jnp.full((npad,), TRASH, jnp.int32)]
    ).reshape(NW, NCHUNK, CHUNK)
    zrows = jnp.zeros((CHUNK, D), jnp.float32)
    zcnt = jnp.zeros((ROWS_PER_TILE,), jnp.float32)
    ones = jnp.ones((CHUNK,), jnp.float32)
    agg_flat, cnt_flat = _sc_aggregate(x, src, dst, zrows, zcnt, ones)
    agg = agg_flat.reshape(NC, N_PAD, D)
    cnt = cnt_flat.reshape(NC, N_PAD, 1)
    return _tc_tail(agg, cnt, x, W_l.T, W_r.T,
                    b_l.reshape(1, D), gamma.reshape(1, D), beta.reshape(1, D))
